# initial kernel scaffold (unmeasured)
import jax
import jax.numpy as jnp
from jax import lax
from jax.experimental import pallas as pl
from jax.experimental.pallas import tpu as pltpu

N_DEV = 8


def kernel(x, w_mat):
    m_per, k = x.shape
    _, n = w_mat.shape
    n_per = n // N_DEV
    m_total = m_per * N_DEV

    def body(x_ref, w_hbm, out_ref, wbuf, ybuf, wsems, send_sems, recv_sems):
        my = lax.axis_index("i")

        def send_desc(h, dst):
            return pltpu.make_async_remote_copy(
                src_ref=ybuf.at[h],
                dst_ref=out_ref.at[pl.ds(my * m_per, m_per), :],
                send_sem=send_sems.at[h],
                recv_sem=recv_sems.at[my],
                device_id=(dst,),
                device_id_type=pl.DeviceIdType.MESH,
            )

        for h in range(N_DEV):
            dst = (my + h) % N_DEV
            cp = pltpu.make_async_copy(
                w_hbm.at[:, pl.ds(dst * n_per, n_per)], wbuf, wsems
            )
            cp.start()
            cp.wait()
            blk = jnp.dot(
                x_ref[:, :], wbuf[:, :], preferred_element_type=jnp.float32
            )
            if h == 0:
                out_ref[pl.ds(my * m_per, m_per), :] = blk
            else:
                ybuf[h, :, :] = blk
                send_desc(h, dst).start()

        for h in range(1, N_DEV):
            dst = (my + h) % N_DEV
            src = (my - h) % N_DEV
            send_desc(h, dst).wait_send()
            recv = pltpu.make_async_remote_copy(
                src_ref=ybuf.at[h],
                dst_ref=out_ref.at[pl.ds(src * m_per, m_per), :],
                send_sem=send_sems.at[h],
                recv_sem=recv_sems.at[src],
                device_id=(dst,),
                device_id_type=pl.DeviceIdType.MESH,
            )
            recv.wait_recv()

    return pl.pallas_call(
        body,
        out_shape=jax.ShapeDtypeStruct((m_total, n_per), jnp.float32),
        in_specs=[
            pl.BlockSpec(memory_space=pltpu.VMEM),
            pl.BlockSpec(memory_space=pltpu.ANY),
        ],
        out_specs=pl.BlockSpec(memory_space=pltpu.VMEM),
        scratch_shapes=[
            pltpu.VMEM((k, n_per), jnp.float32),
            pltpu.VMEM((N_DEV, m_per, n_per), jnp.float32),
            pltpu.SemaphoreType.DMA,
            pltpu.SemaphoreType.DMA((N_DEV,)),
            pltpu.SemaphoreType.DMA((N_DEV,)),
        ],
    )(x, w_mat)


# baseline (device time: 175254 ns/iter reference)
import jax
import jax.numpy as jnp
from jax import lax
from jax.experimental import pallas as pl
from jax.experimental.pallas import tpu as pltpu

N_DEV = 8


def kernel(x, w_mat):
    m_per, k = x.shape
    _, n = w_mat.shape
    n_per = n // N_DEV
    m_total = m_per * N_DEV

    def body(x_ref, w_hbm, out_ref, wbuf, ybuf, wsems, send_sems, recv_sems):
        my = lax.axis_index("i")

        def send_desc(h, dst):
            return pltpu.make_async_remote_copy(
                src_ref=ybuf.at[h],
                dst_ref=out_ref.at[pl.ds(my * m_per, m_per), :],
                send_sem=send_sems.at[h],
                recv_sem=recv_sems.at[my],
                device_id=(dst,),
                device_id_type=pl.DeviceIdType.MESH,
            )

        for h in range(N_DEV):
            dst = (my + h) % N_DEV
            cp = pltpu.make_async_copy(
                w_hbm.at[:, pl.ds(dst * n_per, n_per)], wbuf, wsems
            )
            cp.start()
            cp.wait()
            blk = jnp.dot(
                x_ref[:, :], wbuf[:, :], preferred_element_type=jnp.float32
            )
            if h == 0:
                out_ref[pl.ds(my * m_per, m_per), :] = blk
            else:
                ybuf[h, :, :] = blk
                send_desc(h, dst).start()

        for h in range(1, N_DEV):
            dst = (my + h) % N_DEV
            src = (my - h) % N_DEV
            send_desc(h, dst).wait_send()
            recv = pltpu.make_async_remote_copy(
                src_ref=ybuf.at[h],
                dst_ref=out_ref.at[pl.ds(src * m_per, m_per), :],
                send_sem=send_sems.at[h],
                recv_sem=recv_sems.at[src],
                device_id=(dst,),
                device_id_type=pl.DeviceIdType.MESH,
            )
            recv.wait_recv()

    return pl.pallas_call(
        body,
        out_shape=jax.ShapeDtypeStruct((m_total, n_per), jnp.float32),
        in_specs=[
            pl.BlockSpec(memory_space=pltpu.VMEM),
            pl.BlockSpec(memory_space=pl.ANY),
        ],
        out_specs=pl.BlockSpec(memory_space=pltpu.VMEM),
        scratch_shapes=[
            pltpu.VMEM((k, n_per), jnp.float32),
            pltpu.VMEM((N_DEV, m_per, n_per), jnp.float32),
            pltpu.SemaphoreType.DMA,
            pltpu.SemaphoreType.DMA((N_DEV,)),
            pltpu.SemaphoreType.DMA((N_DEV,)),
        ],
        compiler_params=pltpu.CompilerParams(
            vmem_limit_bytes=100 * 1024 * 1024,
        ),
    )(x, w_mat)


# device time: 170532 ns/iter; 1.0277x vs baseline; 1.0277x over previous
import jax
import jax.numpy as jnp
from jax import lax
from jax.experimental import pallas as pl
from jax.experimental.pallas import tpu as pltpu

N_DEV = 8


def kernel(x, w_mat):
    m_per, k = x.shape
    _, n = w_mat.shape
    n_per = n // N_DEV
    m_total = m_per * N_DEV

    def body(x_ref, w_hbm, out_ref, wbuf, ybuf, wsems, send_sems, recv_sems):
        my = lax.axis_index("i")

        def send_desc(h, dst):
            return pltpu.make_async_remote_copy(
                src_ref=ybuf.at[h],
                dst_ref=out_ref.at[pl.ds(my * m_per, m_per), :],
                send_sem=send_sems.at[h],
                recv_sem=recv_sems.at[my],
                device_id=(dst,),
                device_id_type=pl.DeviceIdType.MESH,
            )

        chunk = n_per // 2

        def wdma(idx, h, half):
            dst = (my + h) % N_DEV
            col = dst * n_per + half * chunk
            return pltpu.make_async_copy(
                w_hbm.at[:, pl.ds(col, chunk)],
                wbuf.at[idx % 2],
                wsems.at[idx % 2],
            )

        order = [
            (h, half) for h in list(range(1, N_DEV)) + [0] for half in range(2)
        ]

        wdma(0, *order[0]).start()
        for idx, (h, half) in enumerate(order):
            dst = (my + h) % N_DEV
            wdma(idx, h, half).wait()
            if idx + 1 < len(order):
                wdma(idx + 1, *order[idx + 1]).start()
            blk = jnp.dot(
                x_ref[:, :], wbuf[idx % 2], preferred_element_type=jnp.float32
            )
            lo, hi = half * chunk, (half + 1) * chunk
            if h == 0:
                out_ref[pl.ds(my * m_per, m_per), lo:hi] = blk
            else:
                ybuf[h, :, lo:hi] = blk
                if half == 1:
                    send_desc(h, dst).start()

        for h in range(1, N_DEV):
            dst = (my + h) % N_DEV
            src = (my - h) % N_DEV
            send_desc(h, dst).wait_send()
            recv = pltpu.make_async_remote_copy(
                src_ref=ybuf.at[h],
                dst_ref=out_ref.at[pl.ds(src * m_per, m_per), :],
                send_sem=send_sems.at[h],
                recv_sem=recv_sems.at[src],
                device_id=(dst,),
                device_id_type=pl.DeviceIdType.MESH,
            )
            recv.wait_recv()

    return pl.pallas_call(
        body,
        out_shape=jax.ShapeDtypeStruct((m_total, n_per), jnp.float32),
        in_specs=[
            pl.BlockSpec(memory_space=pltpu.VMEM),
            pl.BlockSpec(memory_space=pl.ANY),
        ],
        out_specs=pl.BlockSpec(memory_space=pltpu.VMEM),
        scratch_shapes=[
            pltpu.VMEM((2, k, n_per // 2), jnp.float32),
            pltpu.VMEM((N_DEV, m_per, n_per), jnp.float32),
            pltpu.SemaphoreType.DMA((2,)),
            pltpu.SemaphoreType.DMA((N_DEV,)),
            pltpu.SemaphoreType.DMA((N_DEV,)),
        ],
        compiler_params=pltpu.CompilerParams(
            vmem_limit_bytes=100 * 1024 * 1024,
        ),
    )(x, w_mat)


# device time: 153617 ns/iter; 1.1409x vs baseline; 1.1101x over previous
import jax
import jax.numpy as jnp
from jax import lax
from jax.experimental import pallas as pl
from jax.experimental.pallas import tpu as pltpu

N_DEV = 8
ORDER = [1, 3, 4, 2, 5, 7, 6, 0]


def kernel(x, w_mat):
    m_per, k = x.shape
    _, n = w_mat.shape
    n_per = n // N_DEV
    m_total = m_per * N_DEV

    def body(x_ref, w_hbm, out_ref, wbuf, ybuf, wsems, send_sems, recv_sems):
        my = lax.axis_index("i")

        def send_desc(r):
            dst = my ^ r
            return pltpu.make_async_remote_copy(
                src_ref=ybuf.at[r],
                dst_ref=out_ref.at[pl.ds(my * m_per, m_per), :],
                send_sem=send_sems.at[r],
                recv_sem=recv_sems.at[my],
                device_id=(dst,),
                device_id_type=pl.DeviceIdType.MESH,
            )

        chunk = n_per // 2

        def wdma(idx, r, half):
            dst = my ^ r
            col = dst * n_per + half * chunk
            return pltpu.make_async_copy(
                w_hbm.at[:, pl.ds(col, chunk)],
                wbuf.at[idx % 2],
                wsems.at[idx % 2],
            )

        order = [(r, half) for r in ORDER for half in range(2)]

        wdma(0, *order[0]).start()
        for idx, (r, half) in enumerate(order):
            wdma(idx, r, half).wait()
            if idx + 1 < len(order):
                wdma(idx + 1, *order[idx + 1]).start()
            blk = jnp.dot(
                x_ref[:, :], wbuf[idx % 2], preferred_element_type=jnp.float32
            )
            lo, hi = half * chunk, (half + 1) * chunk
            if r == 0:
                out_ref[pl.ds(my * m_per, m_per), lo:hi] = blk
            else:
                ybuf[r, :, lo:hi] = blk
                if half == 1:
                    send_desc(r).start()

        for r in ORDER[:-1]:
            src = my ^ r
            send_desc(r).wait_send()
            recv = pltpu.make_async_remote_copy(
                src_ref=ybuf.at[r],
                dst_ref=out_ref.at[pl.ds(src * m_per, m_per), :],
                send_sem=send_sems.at[r],
                recv_sem=recv_sems.at[src],
                device_id=(my ^ r,),
                device_id_type=pl.DeviceIdType.MESH,
            )
            recv.wait_recv()

    return pl.pallas_call(
        body,
        out_shape=jax.ShapeDtypeStruct((m_total, n_per), jnp.float32),
        in_specs=[
            pl.BlockSpec(memory_space=pltpu.VMEM),
            pl.BlockSpec(memory_space=pl.ANY),
        ],
        out_specs=pl.BlockSpec(memory_space=pltpu.VMEM),
        scratch_shapes=[
            pltpu.VMEM((2, k, n_per // 2), jnp.float32),
            pltpu.VMEM((N_DEV, m_per, n_per), jnp.float32),
            pltpu.SemaphoreType.DMA((2,)),
            pltpu.SemaphoreType.DMA((N_DEV,)),
            pltpu.SemaphoreType.DMA((N_DEV,)),
        ],
        compiler_params=pltpu.CompilerParams(
            vmem_limit_bytes=100 * 1024 * 1024,
        ),
    )(x, w_mat)
